# Initial kernel scaffold; baseline (speedup 1.0000x reference)
import jax
import jax.numpy as jnp
from jax.experimental import pallas as pl


def kernel(x, synth_emb_weight):
    raise NotImplementedError("write your pallas kernel here")


# speedup vs baseline: 3.9162x; 3.9162x over previous
import functools

import jax
import jax.numpy as jnp
from jax import lax
from jax.experimental import pallas as pl
from jax.experimental.pallas import tpu as pltpu
from jax.experimental.pallas import tpu_sc as plsc

EMB_DIM = 64
BATCH = 16384
SEQ = 50
B_ROWS = BATCH * SEQ
PAIRS = B_ROWS // 2
NC = 2
NS = 16
NW = NC * NS
PPW = PAIRS // NW
CH = 128
CPW = PPW // CH


def _build_pair_table(w_ref, pt_ref):
    i = lax.broadcasted_iota(jnp.int32, (64, 1), 0)
    i0 = i >> 3
    i1 = i & 7
    left = jnp.zeros((64, EMB_DIM), jnp.float32)
    right = jnp.zeros((64, EMB_DIM), jnp.float32)
    for v in range(7):
        row = w_ref[pl.ds(v, 1), :]
        left = left + jnp.where(i0 == v, 1.0, 0.0) * row
        right = right + jnp.where(i1 == v, 1.0, 0.0) * row
    pt_ref[:, 0:EMB_DIM] = left
    pt_ref[:, EMB_DIM:2 * EMB_DIM] = right


_pair_table = pl.pallas_call(
    _build_pair_table,
    out_shape=jax.ShapeDtypeStruct((64, 2 * EMB_DIM), jnp.float32),
)


def _pair_index_kernel(x_ref, o_ref):
    xf = x_ref[...].astype(jnp.float32)
    even = lax.broadcasted_iota(jnp.int32, (1, 128), 1) % 2 == 0
    y = xf * jnp.where(even, 8.0, 1.0)
    s = (lax.broadcasted_iota(jnp.int32, (128, 64), 0) >> 1
         == lax.broadcasted_iota(jnp.int32, (128, 64), 1)).astype(jnp.float32)
    p = lax.dot_general(y, s, (((1,), (0,)), ((), ())),
                        preferred_element_type=jnp.float32)
    o_ref[...] = p.astype(jnp.int32)


_pair_index = pl.pallas_call(
    _pair_index_kernel,
    out_shape=jax.ShapeDtypeStruct((B_ROWS // 128, 64), jnp.int32),
)

_mesh = plsc.VectorSubcoreMesh(core_axis_name="c", subcore_axis_name="s")


@functools.partial(
    pl.kernel,
    out_type=jax.ShapeDtypeStruct((PAIRS, 2 * EMB_DIM), jnp.float32),
    mesh=_mesh,
    scratch_types=[
        pltpu.VMEM((CPW, CH), jnp.int32),
        pltpu.VMEM((CH, 2 * EMB_DIM), jnp.float32),
        pltpu.SemaphoreType.DMA,
    ],
)
def _emb_gather(pidx_hbm, pt_hbm, out_hbm, pidxv, rows, sem):
    wid = lax.axis_index("s") * NC + lax.axis_index("c")
    pltpu.sync_copy(pidx_hbm.at[wid], pidxv)

    def chunk(c, carry):
        pltpu.async_copy(pt_hbm.at[pidxv.at[c]], rows, sem).wait()
        base = pl.multiple_of(wid * PPW + c * CH, CH)
        pltpu.sync_copy(rows, out_hbm.at[pl.ds(base, CH)])
        return carry

    lax.fori_loop(0, CPW, chunk, 0)


def kernel(x, synth_emb_weight):
    pt = _pair_table(synth_emb_weight)
    pidx = _pair_index(x.reshape(B_ROWS // 128, 128).astype(jnp.int32))
    out = _emb_gather(pidx.reshape(NW, CPW, CH), pt)
    return out.reshape(BATCH, SEQ * EMB_DIM)


# speedup vs baseline: 3.9596x; 1.0111x over previous
import functools

import jax
import jax.numpy as jnp
from jax import lax
from jax.experimental import pallas as pl
from jax.experimental.pallas import tpu as pltpu
from jax.experimental.pallas import tpu_sc as plsc

EMB_DIM = 64
BATCH = 16384
SEQ = 50
B_ROWS = BATCH * SEQ
PAIRS = B_ROWS // 2
NC = 2
NS = 16
NW = NC * NS
PPW = PAIRS // NW
CH = 128
CPW = PPW // CH


def _build_pair_table(w_ref, pt_ref):
    i = lax.broadcasted_iota(jnp.int32, (64, 1), 0)
    i0 = i >> 3
    i1 = i & 7
    left = jnp.zeros((64, EMB_DIM), jnp.float32)
    right = jnp.zeros((64, EMB_DIM), jnp.float32)
    for v in range(7):
        row = w_ref[pl.ds(v, 1), :]
        left = left + jnp.where(i0 == v, 1.0, 0.0) * row
        right = right + jnp.where(i1 == v, 1.0, 0.0) * row
    pt_ref[:, 0:EMB_DIM] = left
    pt_ref[:, EMB_DIM:2 * EMB_DIM] = right


_pair_table = pl.pallas_call(
    _build_pair_table,
    out_shape=jax.ShapeDtypeStruct((64, 2 * EMB_DIM), jnp.float32),
)


def _pair_index_kernel(x_ref, o_ref):
    xf = x_ref[...].astype(jnp.float32)
    even = lax.broadcasted_iota(jnp.int32, (1, 128), 1) % 2 == 0
    y = xf * jnp.where(even, 8.0, 1.0)
    s = (lax.broadcasted_iota(jnp.int32, (128, 64), 0) >> 1
         == lax.broadcasted_iota(jnp.int32, (128, 64), 1)).astype(jnp.float32)
    p = lax.dot_general(y, s, (((1,), (0,)), ((), ())),
                        preferred_element_type=jnp.float32)
    o_ref[...] = p.astype(jnp.int32)


_pair_index = pl.pallas_call(
    _pair_index_kernel,
    out_shape=jax.ShapeDtypeStruct((B_ROWS // 128, 64), jnp.int32),
)

_mesh = plsc.VectorSubcoreMesh(core_axis_name="c", subcore_axis_name="s")


@functools.partial(
    pl.kernel,
    out_type=jax.ShapeDtypeStruct((PAIRS, 2 * EMB_DIM), jnp.float32),
    mesh=_mesh,
    scratch_types=[
        pltpu.VMEM((CPW, CH), jnp.int32),
        pltpu.VMEM((CH, 2 * EMB_DIM), jnp.float32),
        pltpu.VMEM((CH, 2 * EMB_DIM), jnp.float32),
        pltpu.SemaphoreType.DMA,
        pltpu.SemaphoreType.DMA,
        pltpu.SemaphoreType.DMA,
        pltpu.SemaphoreType.DMA,
    ],
)
def _emb_gather(pidx_hbm, pt_hbm, out_hbm, pidxv, rows_a, rows_b,
                sg_a, sg_b, sw_a, sw_b):
    wid = lax.axis_index("s") * NC + lax.axis_index("c")
    pltpu.sync_copy(pidx_hbm.at[wid], pidxv)

    def gather(c, buf, sem):
        pltpu.async_copy(pt_hbm.at[pidxv.at[c]], buf, sem)

    def write(c, buf, sem):
        base = pl.multiple_of(wid * PPW + c * CH, CH)
        pltpu.async_copy(buf, out_hbm.at[pl.ds(base, CH)], sem)

    def drain(buf, sem):
        pltpu.make_async_copy(buf, out_hbm.at[pl.ds(0, CH)], sem).wait()

    gather(0, rows_a, sg_a)

    def body(i, carry):
        e = 2 * i
        o = e + 1

        @pl.when(i > 0)
        def _():
            drain(rows_b, sw_b)
        gather(o, rows_b, sg_b)
        drain(rows_a, sg_a)
        write(e, rows_a, sw_a)
        drain(rows_a, sw_a)
        @pl.when(i + 1 < CPW // 2)
        def _():
            gather(e + 2, rows_a, sg_a)
        drain(rows_b, sg_b)
        write(o, rows_b, sw_b)
        return carry

    lax.fori_loop(0, CPW // 2, body, 0)
    drain(rows_b, sw_b)


def kernel(x, synth_emb_weight):
    pt = _pair_table(synth_emb_weight)
    pidx = _pair_index(x.reshape(B_ROWS // 128, 128).astype(jnp.int32))
    out = _emb_gather(pidx.reshape(NW, CPW, CH), pt)
    return out.reshape(BATCH, SEQ * EMB_DIM)
